# baseline (speedup 1.0000x reference)
import functools

import jax
import jax.numpy as jnp
from jax import lax
from jax.experimental import pallas as pl
from jax.experimental.pallas import tpu as pltpu
from jax.experimental.pallas import tpu_sc as plsc

N = 10000
E = 160000
HEADS = 4

NC, NS, L = 2, 16, 16
KB = 128
E2 = -(-E // (2 * NS * KB)) * 2 * NS * KB


BLK = 1000


def _leaky(x):
    return jnp.where(x > 0, x, x * jnp.float32(0.2))


def _proj(x, W, a_src, a_dst):
    H, out_c = a_src.shape
    K = x.shape[1]
    C = H * out_c // 128

    def body(x_ref, w_ref, as_ref, ad_ref, hc_ref, al_s_ref, al_d_ref,
             exs_ref):
        h = jnp.dot(x_ref[...], w_ref[...],
                    preferred_element_type=jnp.float32)
        for c in range(C):
            hc_ref[c] = h[:, c * 128:(c + 1) * 128]
        als = []
        ald = []
        for hd in range(H):
            blkh = h[:, hd * out_c:(hd + 1) * out_c]
            als.append((blkh * as_ref[hd][None, :]).sum(-1, keepdims=True))
            ald.append((blkh * ad_ref[hd][None, :]).sum(-1, keepdims=True))
        als = jnp.concatenate(als, axis=1)
        ald = jnp.concatenate(ald, axis=1)
        al_s_ref[...] = als
        al_d_ref[...] = ald
        exs_ref[...] = jnp.exp(_leaky(als + ald))

    return pl.pallas_call(
        body,
        grid=(N // BLK,),
        in_specs=[
            pl.BlockSpec((BLK, K), lambda i: (i, 0)),
            pl.BlockSpec((K, H * out_c), lambda i: (0, 0)),
            pl.BlockSpec((H, out_c), lambda i: (0, 0)),
            pl.BlockSpec((H, out_c), lambda i: (0, 0)),
        ],
        out_specs=[
            pl.BlockSpec((C, BLK, 128), lambda i: (0, i, 0)),
            pl.BlockSpec((BLK, H), lambda i: (i, 0)),
            pl.BlockSpec((BLK, H), lambda i: (i, 0)),
            pl.BlockSpec((BLK, H), lambda i: (i, 0)),
        ],
        out_shape=[
            jax.ShapeDtypeStruct((C, N, 128), jnp.float32),
            jax.ShapeDtypeStruct((N, H), jnp.float32),
            jax.ShapeDtypeStruct((N, H), jnp.float32),
            jax.ShapeDtypeStruct((N, H), jnp.float32),
        ],
    )(x, W, a_src, a_dst)


def _mid(acc, hc, dent, exs, b1, W2, a_src2, a_dst2):
    C1 = acc.shape[0]
    H2, out_c2 = a_src2.shape
    C2 = H2 * out_c2 // 128

    def body(acc_ref, hc_ref, dent_ref, exs_ref, b1_ref, w2_ref, as2_ref,
             ad2_ref, hc2_ref, al_s_ref, al_d_ref, exs2_ref):
        cols = []
        for c in range(C1):
            hd = c // 2
            col = acc_ref[c] + exs_ref[...][:, hd:hd + 1] * hc_ref[c]
            col = col / (dent_ref[...][:, hd:hd + 1] + jnp.float32(1e-16))
            col = col + b1_ref[...][:, c * 128:(c + 1) * 128]
            cols.append(col)
        hrow = jnp.concatenate(cols, axis=1)
        hrow = jnp.where(hrow > 0, hrow, jnp.exp(hrow) - jnp.float32(1.0))
        h2 = jnp.dot(hrow, w2_ref[...], preferred_element_type=jnp.float32)
        for c in range(C2):
            hc2_ref[c] = h2[:, c * 128:(c + 1) * 128]
        als = []
        ald = []
        for hd in range(H2):
            blkh = h2[:, hd * out_c2:(hd + 1) * out_c2]
            als.append((blkh * as2_ref[hd][None, :]).sum(-1, keepdims=True))
            ald.append((blkh * ad2_ref[hd][None, :]).sum(-1, keepdims=True))
        als = jnp.concatenate(als, axis=1) if H2 > 1 else als[0]
        ald = jnp.concatenate(ald, axis=1) if H2 > 1 else ald[0]
        al_s_ref[...] = als
        al_d_ref[...] = ald
        exs2_ref[...] = jnp.exp(_leaky(als + ald))

    return pl.pallas_call(
        body,
        grid=(N // BLK,),
        in_specs=[
            pl.BlockSpec((C1, BLK, 128), lambda i: (0, i, 0)),
            pl.BlockSpec((C1, BLK, 128), lambda i: (0, i, 0)),
            pl.BlockSpec((BLK, C1 // 2), lambda i: (i, 0)),
            pl.BlockSpec((BLK, C1 // 2), lambda i: (i, 0)),
            pl.BlockSpec((1, C1 * 128), lambda i: (0, 0)),
            pl.BlockSpec((C1 * 128, H2 * out_c2), lambda i: (0, 0)),
            pl.BlockSpec((H2, out_c2), lambda i: (0, 0)),
            pl.BlockSpec((H2, out_c2), lambda i: (0, 0)),
        ],
        out_specs=[
            pl.BlockSpec((C2, BLK, 128), lambda i: (0, i, 0)),
            pl.BlockSpec((BLK, H2), lambda i: (i, 0)),
            pl.BlockSpec((BLK, H2), lambda i: (i, 0)),
            pl.BlockSpec((BLK, H2), lambda i: (i, 0)),
        ],
        out_shape=[
            jax.ShapeDtypeStruct((C2, N, 128), jnp.float32),
            jax.ShapeDtypeStruct((N, H2), jnp.float32),
            jax.ShapeDtypeStruct((N, H2), jnp.float32),
            jax.ShapeDtypeStruct((N, H2), jnp.float32),
        ],
    )(acc, hc, dent, exs, b1.reshape(1, -1), W2, a_src2, a_dst2)


def _final(acc2, hc2, dent2, exs2, b2):
    C2 = acc2.shape[0]

    def body(acc_ref, hc_ref, dent_ref, exs_ref, b2_ref, o_ref):
        cols = []
        for c in range(C2):
            col = acc_ref[c] + exs_ref[...] * hc_ref[c]
            col = col / (dent_ref[...] + jnp.float32(1e-16))
            cols.append(col + b2_ref[...][:, c * 128:(c + 1) * 128])
        o_ref[...] = jnp.concatenate(cols, axis=1)

    return pl.pallas_call(
        body,
        grid=(N // BLK,),
        in_specs=[
            pl.BlockSpec((C2, BLK, 128), lambda i: (0, i, 0)),
            pl.BlockSpec((C2, BLK, 128), lambda i: (0, i, 0)),
            pl.BlockSpec((BLK, 1), lambda i: (i, 0)),
            pl.BlockSpec((BLK, 1), lambda i: (i, 0)),
            pl.BlockSpec((1, C2 * 128), lambda i: (0, 0)),
        ],
        out_specs=pl.BlockSpec((BLK, C2 * 128), lambda i: (i, 0)),
        out_shape=jax.ShapeDtypeStruct((N, C2 * 128), jnp.float32),
    )(acc2, hc2, dent2, exs2, b2.reshape(1, -1))


def _build_edge_agg(C):
    CPS = C // NC
    EB = E2 // NS
    NB = EB // KB
    NBH = NB // 2
    RW = N // NS // 8 * 8
    mesh = plsc.VectorSubcoreMesh(core_axis_name="c", subcore_axis_name="s",
                                  num_cores=NC, num_subcores=NS)

    @functools.partial(
        pl.kernel,
        out_type=jax.ShapeDtypeStruct((C, N, 128), jnp.float32),
        mesh=mesh,
        compiler_params=pltpu.CompilerParams(needs_layout_passes=False),
        scratch_types=[
            pltpu.VMEM_SHARED((N, 128), jnp.float32),
            pltpu.VMEM((NBH, KB), jnp.int32),
            pltpu.VMEM((NBH, KB), jnp.int32),
            pltpu.VMEM((NBH * KB,), jnp.float32),
            pltpu.VMEM((KB, 128), jnp.float32),
            pltpu.VMEM((KB, 128), jnp.float32),
            pltpu.SemaphoreType.DMA,
            pltpu.SemaphoreType.DMA,
            pltpu.SemaphoreType.DMA,
            pltpu.SemaphoreType.DMA,
        ],
    )
    def agg(hc, src2, dst2, ex2, out, acc_sp, src_v, dst_v, ex_v,
            g0, g1, sem0, sem1, ses0, ses1):
        cid = lax.axis_index("c")
        sid = lax.axis_index("s")
        wbase = sid * RW
        last = sid == NS - 1

        for j in range(CPS):
            chunk = cid * CPS + j
            hd = chunk // 2

            @plsc.parallel_loop(0, KB, unroll=4)
            def zrow(r):
                for c16 in range(128 // L):
                    g0[r, pl.ds(c16 * L, L)] = jnp.zeros((L,), jnp.float32)
            for k in range(RW // KB):
                pltpu.sync_copy(g0, acc_sp.at[pl.ds(wbase + k * KB, KB)])
            pltpu.sync_copy(g0.at[pl.ds(0, RW % KB)],
                            acc_sp.at[pl.ds(wbase + RW - RW % KB, RW % KB)])
            @pl.when(last)
            def _():
                pltpu.sync_copy(g0.at[pl.ds(0, N - RW * NS)],
                                acc_sp.at[pl.ds(RW * NS, N - RW * NS)])
            plsc.subcore_barrier()

            for half in range(2):
                pltpu.sync_copy(src2.at[sid, pl.ds(half * NBH, NBH)], src_v)
                pltpu.sync_copy(dst2.at[sid, pl.ds(half * NBH, NBH)], dst_v)
                exoff = pl.multiple_of(
                    hd * E2 + sid * EB + half * NBH * KB, 8)
                pltpu.sync_copy(ex2.at[pl.ds(exoff, NBH * KB)], ex_v)

                pltpu.async_copy(hc.at[chunk].at[src_v.at[0]], g0, sem0)
                pltpu.async_copy(hc.at[chunk].at[src_v.at[1]], g1, sem1)

                bufs = ((g0, sem0, ses0), (g1, sem1, ses1))

                def pair(ph, _):
                    more = ph < NBH // 2 - 1
                    for par, (g, sem, ses) in enumerate(bufs):
                        b = 2 * ph + par
                        pltpu.make_async_copy(
                            hc.at[chunk].at[src_v.at[b]], g, sem).wait()

                        @plsc.parallel_loop(0, KB, unroll=4)
                        def mrow(r):
                            w = plsc.load_gather(
                                ex_v,
                                [jnp.full((L,), b * KB + r, jnp.int32)])
                            for c16 in range(128 // L):
                                g[r, pl.ds(c16 * L, L)] = (
                                    g[r, pl.ds(c16 * L, L)] * w)

                        pltpu.async_copy(g, acc_sp.at[dst_v.at[b]], ses,
                                         add=True)
                    for par, (g, sem, ses) in enumerate(bufs):
                        b = 2 * ph + par

                        @pl.when(more)
                        def _():
                            pltpu.make_async_copy(
                                g, acc_sp.at[dst_v.at[b]], ses).wait()
                            pltpu.async_copy(
                                hc.at[chunk].at[src_v.at[b + 2]], g, sem)
                    return 0
                lax.fori_loop(0, NBH // 2, pair, 0)
                for par, (g, sem, ses) in enumerate(bufs):
                    b = NBH - 2 + par
                    pltpu.make_async_copy(
                        g, acc_sp.at[dst_v.at[b]], ses).wait()

            plsc.subcore_barrier()

            for k in range(RW // KB):
                pltpu.sync_copy(acc_sp.at[pl.ds(wbase + k * KB, KB)],
                                out.at[chunk, pl.ds(wbase + k * KB, KB)])
            pltpu.sync_copy(
                acc_sp.at[pl.ds(wbase + RW - RW % KB, RW % KB)],
                out.at[chunk, pl.ds(wbase + RW - RW % KB, RW % KB)])
            @pl.when(last)
            def _():
                pltpu.sync_copy(acc_sp.at[pl.ds(RW * NS, N - RW * NS)],
                                out.at[chunk, pl.ds(RW * NS, N - RW * NS)])

    return agg


_edge_agg = {c: _build_edge_agg(c) for c in (8, 2)}


def _build_edge_attn(H):
    HPS = max(H // NC, 1)
    EB = E2 // NS
    G = EB // L
    mesh = plsc.VectorSubcoreMesh(core_axis_name="c", subcore_axis_name="s",
                                  num_cores=NC, num_subcores=NS)
    scr = ([pltpu.VMEM((N,), jnp.float32)] * (3 * HPS)
           + [pltpu.VMEM((EB,), jnp.int32)] * 2
           + [pltpu.VMEM((EB,), jnp.float32)] * HPS)

    @functools.partial(
        pl.kernel,
        out_type=(jax.ShapeDtypeStruct((H * E2,), jnp.float32),
                  jax.ShapeDtypeStruct((H * NS * N,), jnp.float32)),
        mesh=mesh,
        compiler_params=pltpu.CompilerParams(needs_layout_passes=False),
        scratch_types=scr,
    )
    def attn(asrc_f, adst_f, srcf, dstf, exw, denom_f, *scratch):
        as_t = scratch[0:HPS]
        ad_t = scratch[HPS:2 * HPS]
        den = scratch[2 * HPS:3 * HPS]
        src_v, dst_v = scratch[3 * HPS], scratch[3 * HPS + 1]
        exb = scratch[3 * HPS + 2:]
        cid = lax.axis_index("c")
        sid = lax.axis_index("s")

        for j in range(HPS):
            h = cid * HPS + j if H > 1 else 0
            off = pl.multiple_of(h * N, 8)
            pltpu.sync_copy(asrc_f.at[pl.ds(off, N)], as_t[j])
            pltpu.sync_copy(adst_f.at[pl.ds(off, N)], ad_t[j])
        ebase = sid * EB
        pltpu.sync_copy(srcf.at[pl.ds(ebase, EB)], src_v)
        pltpu.sync_copy(dstf.at[pl.ds(ebase, EB)], dst_v)

        def zero(i, _):
            for j in range(HPS):
                den[j][pl.ds(i * L, L)] = jnp.zeros((L,), jnp.float32)
            return 0
        lax.fori_loop(0, N // L, zero, 0)

        def grp(g, _):
            sv = src_v[pl.ds(g * L, L)]
            dv = dst_v[pl.ds(g * L, L)]
            ge = ebase + g * L + lax.iota(jnp.int32, L)
            valid = ge < E
            for j in range(HPS):
                a = plsc.load_gather(as_t[j], [sv])
                b = plsc.load_gather(ad_t[j], [dv])
                al = a + b
                al = jnp.where(al > 0, al, al * jnp.float32(0.2))
                e = jnp.where(valid, jnp.exp(al), jnp.float32(0.0))
                exb[j][pl.ds(g * L, L)] = e
                plsc.addupdate_scatter(den[j], [dv], e)
            return 0
        lax.fori_loop(0, G, grp, 0)

        def write():
            for j in range(HPS):
                h = cid * HPS + j if H > 1 else 0
                off = pl.multiple_of(h * E2 + ebase, 8)
                pltpu.sync_copy(exb[j], exw.at[pl.ds(off, EB)])
                doff = pl.multiple_of((h * NS + sid) * N, 8)
                pltpu.sync_copy(den[j], denom_f.at[pl.ds(doff, N)])
        if H == 1:
            pl.when(cid == 0)(write)
        else:
            write()

    return attn


_edge_attn = {h: _build_edge_attn(h) for h in (4, 1)}


def _attn_and_agg(hc, als, ald, srcf, dstf, src2, dst2, exs, heads):
    n = als.shape[0]
    exw, denom_f = _edge_attn[heads](
        als.T.reshape(heads * n), ald.T.reshape(heads * n), srcf, dstf)
    acc = _edge_agg[hc.shape[0]](hc, src2, dst2, exw)
    dent = denom_f.reshape(heads, NS, n).sum(axis=1).T + exs
    return acc, dent


def kernel(x, edge_index, W1, a_src1, a_dst1, b1, W2, a_src2, a_dst2, b2):
    src, dst = edge_index[0], edge_index[1]
    pad = E2 - E
    nb = E2 // NS // KB
    srcf = jnp.pad(src, (0, pad))
    dstf = jnp.pad(dst, (0, pad))
    src2 = srcf.reshape(NS, nb, KB)
    dst2 = dstf.reshape(NS, nb, KB)

    hc1, als1, ald1, exs1 = _proj(x, W1, a_src1, a_dst1)
    acc1, dent1 = _attn_and_agg(hc1, als1, ald1, srcf, dstf, src2, dst2,
                                exs1, HEADS)
    hc2, als2, ald2, exs2 = _mid(acc1, hc1, dent1, exs1, b1, W2,
                                 a_src2, a_dst2)
    acc2, dent2 = _attn_and_agg(hc2, als2, ald2, srcf, dstf, src2, dst2,
                                exs2, 1)
    return _final(acc2, hc2, dent2, exs2, b2)


# speedup vs baseline: 1.0562x; 1.0562x over previous
import functools

import jax
import jax.numpy as jnp
from jax import lax
from jax.experimental import pallas as pl
from jax.experimental.pallas import tpu as pltpu
from jax.experimental.pallas import tpu_sc as plsc

N = 10000
E = 160000
HEADS = 4

NC, NS, L = 2, 16, 16
KB = 128
E2 = -(-E // (2 * NS * KB)) * 2 * NS * KB


BLK = 1000


def _leaky(x):
    return jnp.where(x > 0, x, x * jnp.float32(0.2))


def _proj(x, W, a_src, a_dst):
    H, out_c = a_src.shape
    K = x.shape[1]
    C = H * out_c // 128

    def body(x_ref, w_ref, as_ref, ad_ref, hc_ref, al_s_ref, al_d_ref,
             exs_ref):
        h = jnp.dot(x_ref[...], w_ref[...],
                    preferred_element_type=jnp.float32)
        for c in range(C):
            hc_ref[c] = h[:, c * 128:(c + 1) * 128]
        als = []
        ald = []
        for hd in range(H):
            blkh = h[:, hd * out_c:(hd + 1) * out_c]
            als.append((blkh * as_ref[hd][None, :]).sum(-1, keepdims=True))
            ald.append((blkh * ad_ref[hd][None, :]).sum(-1, keepdims=True))
        als = jnp.concatenate(als, axis=1)
        ald = jnp.concatenate(ald, axis=1)
        al_s_ref[...] = als
        al_d_ref[...] = ald
        exs_ref[...] = jnp.exp(_leaky(als + ald))

    return pl.pallas_call(
        body,
        grid=(N // BLK,),
        in_specs=[
            pl.BlockSpec((BLK, K), lambda i: (i, 0)),
            pl.BlockSpec((K, H * out_c), lambda i: (0, 0)),
            pl.BlockSpec((H, out_c), lambda i: (0, 0)),
            pl.BlockSpec((H, out_c), lambda i: (0, 0)),
        ],
        out_specs=[
            pl.BlockSpec((C, BLK, 128), lambda i: (0, i, 0)),
            pl.BlockSpec((BLK, H), lambda i: (i, 0)),
            pl.BlockSpec((BLK, H), lambda i: (i, 0)),
            pl.BlockSpec((BLK, H), lambda i: (i, 0)),
        ],
        out_shape=[
            jax.ShapeDtypeStruct((C, N, 128), jnp.float32),
            jax.ShapeDtypeStruct((N, H), jnp.float32),
            jax.ShapeDtypeStruct((N, H), jnp.float32),
            jax.ShapeDtypeStruct((N, H), jnp.float32),
        ],
    )(x, W, a_src, a_dst)


def _mid(acc, hc, dent, exs, b1, W2, a_src2, a_dst2):
    C1 = acc.shape[0]
    H2, out_c2 = a_src2.shape
    C2 = H2 * out_c2 // 128

    def body(acc_ref, hc_ref, dent_ref, exs_ref, b1_ref, w2_ref, as2_ref,
             ad2_ref, hc2_ref, al_s_ref, al_d_ref, exs2_ref):
        cols = []
        for c in range(C1):
            hd = c // 2
            col = acc_ref[c] + exs_ref[...][:, hd:hd + 1] * hc_ref[c]
            col = col / (dent_ref[...][:, hd:hd + 1] + jnp.float32(1e-16))
            col = col + b1_ref[...][:, c * 128:(c + 1) * 128]
            cols.append(col)
        hrow = jnp.concatenate(cols, axis=1)
        hrow = jnp.where(hrow > 0, hrow, jnp.exp(hrow) - jnp.float32(1.0))
        h2 = jnp.dot(hrow, w2_ref[...], preferred_element_type=jnp.float32)
        for c in range(C2):
            hc2_ref[c] = h2[:, c * 128:(c + 1) * 128]
        als = []
        ald = []
        for hd in range(H2):
            blkh = h2[:, hd * out_c2:(hd + 1) * out_c2]
            als.append((blkh * as2_ref[hd][None, :]).sum(-1, keepdims=True))
            ald.append((blkh * ad2_ref[hd][None, :]).sum(-1, keepdims=True))
        als = jnp.concatenate(als, axis=1) if H2 > 1 else als[0]
        ald = jnp.concatenate(ald, axis=1) if H2 > 1 else ald[0]
        al_s_ref[...] = als
        al_d_ref[...] = ald
        exs2_ref[...] = jnp.exp(_leaky(als + ald))

    return pl.pallas_call(
        body,
        grid=(N // BLK,),
        in_specs=[
            pl.BlockSpec((C1, BLK, 128), lambda i: (0, i, 0)),
            pl.BlockSpec((C1, BLK, 128), lambda i: (0, i, 0)),
            pl.BlockSpec((BLK, C1 // 2), lambda i: (i, 0)),
            pl.BlockSpec((BLK, C1 // 2), lambda i: (i, 0)),
            pl.BlockSpec((1, C1 * 128), lambda i: (0, 0)),
            pl.BlockSpec((C1 * 128, H2 * out_c2), lambda i: (0, 0)),
            pl.BlockSpec((H2, out_c2), lambda i: (0, 0)),
            pl.BlockSpec((H2, out_c2), lambda i: (0, 0)),
        ],
        out_specs=[
            pl.BlockSpec((C2, BLK, 128), lambda i: (0, i, 0)),
            pl.BlockSpec((BLK, H2), lambda i: (i, 0)),
            pl.BlockSpec((BLK, H2), lambda i: (i, 0)),
            pl.BlockSpec((BLK, H2), lambda i: (i, 0)),
        ],
        out_shape=[
            jax.ShapeDtypeStruct((C2, N, 128), jnp.float32),
            jax.ShapeDtypeStruct((N, H2), jnp.float32),
            jax.ShapeDtypeStruct((N, H2), jnp.float32),
            jax.ShapeDtypeStruct((N, H2), jnp.float32),
        ],
    )(acc, hc, dent, exs, b1.reshape(1, -1), W2, a_src2, a_dst2)


def _final(acc2, hc2, dent2, exs2, b2):
    C2 = acc2.shape[0]

    def body(acc_ref, hc_ref, dent_ref, exs_ref, b2_ref, o_ref):
        cols = []
        for c in range(C2):
            col = acc_ref[c] + exs_ref[...] * hc_ref[c]
            col = col / (dent_ref[...] + jnp.float32(1e-16))
            cols.append(col + b2_ref[...][:, c * 128:(c + 1) * 128])
        o_ref[...] = jnp.concatenate(cols, axis=1)

    return pl.pallas_call(
        body,
        grid=(N // BLK,),
        in_specs=[
            pl.BlockSpec((C2, BLK, 128), lambda i: (0, i, 0)),
            pl.BlockSpec((C2, BLK, 128), lambda i: (0, i, 0)),
            pl.BlockSpec((BLK, 1), lambda i: (i, 0)),
            pl.BlockSpec((BLK, 1), lambda i: (i, 0)),
            pl.BlockSpec((1, C2 * 128), lambda i: (0, 0)),
        ],
        out_specs=pl.BlockSpec((BLK, C2 * 128), lambda i: (i, 0)),
        out_shape=jax.ShapeDtypeStruct((N, C2 * 128), jnp.float32),
    )(acc2, hc2, dent2, exs2, b2.reshape(1, -1))


def _build_edge_agg(C):
    CPS = C // NC
    EB = E2 // NS
    NB = EB // KB
    NBH = NB // 2
    RW = N // NS // 8 * 8
    mesh = plsc.VectorSubcoreMesh(core_axis_name="c", subcore_axis_name="s",
                                  num_cores=NC, num_subcores=NS)

    @functools.partial(
        pl.kernel,
        out_type=jax.ShapeDtypeStruct((C, N, 128), jnp.float32),
        mesh=mesh,
        compiler_params=pltpu.CompilerParams(needs_layout_passes=False),
        scratch_types=[
            pltpu.VMEM_SHARED((N, 128), jnp.float32),
            pltpu.VMEM((NBH, KB), jnp.int32),
            pltpu.VMEM((NBH, KB), jnp.int32),
            pltpu.VMEM((NBH * KB,), jnp.float32),
            pltpu.VMEM((KB, 128), jnp.float32),
            pltpu.VMEM((KB, 128), jnp.float32),
            pltpu.SemaphoreType.DMA,
            pltpu.SemaphoreType.DMA,
        ],
    )
    def agg(hc, src2, dst2, ex2, out, acc_sp, src_v, dst_v, ex_v,
            g0, g1, sem0, sem1):
        cid = lax.axis_index("c")
        sid = lax.axis_index("s")
        wbase = sid * RW
        last = sid == NS - 1

        for j in range(CPS):
            chunk = cid * CPS + j
            hd = chunk // 2

            @plsc.parallel_loop(0, KB, unroll=4)
            def zrow(r):
                for c16 in range(128 // L):
                    g0[r, pl.ds(c16 * L, L)] = jnp.zeros((L,), jnp.float32)
            for k in range(RW // KB):
                pltpu.sync_copy(g0, acc_sp.at[pl.ds(wbase + k * KB, KB)])
            pltpu.sync_copy(g0.at[pl.ds(0, RW % KB)],
                            acc_sp.at[pl.ds(wbase + RW - RW % KB, RW % KB)])
            @pl.when(last)
            def _():
                pltpu.sync_copy(g0.at[pl.ds(0, N - RW * NS)],
                                acc_sp.at[pl.ds(RW * NS, N - RW * NS)])
            plsc.subcore_barrier()

            for half in range(2):
                pltpu.sync_copy(src2.at[sid, pl.ds(half * NBH, NBH)], src_v)
                pltpu.sync_copy(dst2.at[sid, pl.ds(half * NBH, NBH)], dst_v)
                exoff = pl.multiple_of(
                    hd * E2 + sid * EB + half * NBH * KB, 8)
                pltpu.sync_copy(ex2.at[pl.ds(exoff, NBH * KB)], ex_v)

                pltpu.async_copy(hc.at[chunk].at[src_v.at[0]], g0, sem0)
                pltpu.async_copy(hc.at[chunk].at[src_v.at[1]], g1, sem1)

                def pair(ph, _):
                    more = ph < NBH // 2 - 1
                    for par, (g, sem) in enumerate(((g0, sem0),
                                                    (g1, sem1))):
                        b = 2 * ph + par
                        pltpu.make_async_copy(
                            hc.at[chunk].at[src_v.at[b]], g, sem).wait()

                        @plsc.parallel_loop(0, KB, unroll=4)
                        def mrow(r):
                            w = plsc.load_gather(
                                ex_v,
                                [jnp.full((L,), b * KB + r, jnp.int32)])
                            for c16 in range(128 // L):
                                g[r, pl.ds(c16 * L, L)] = (
                                    g[r, pl.ds(c16 * L, L)] * w)

                        pltpu.sync_copy(g, acc_sp.at[dst_v.at[b]], add=True)

                        @pl.when(more)
                        def _():
                            pltpu.async_copy(
                                hc.at[chunk].at[src_v.at[b + 2]], g, sem)
                    return 0
                lax.fori_loop(0, NBH // 2, pair, 0)

            plsc.subcore_barrier()

            for k in range(RW // KB):
                pltpu.sync_copy(acc_sp.at[pl.ds(wbase + k * KB, KB)],
                                out.at[chunk, pl.ds(wbase + k * KB, KB)])
            pltpu.sync_copy(
                acc_sp.at[pl.ds(wbase + RW - RW % KB, RW % KB)],
                out.at[chunk, pl.ds(wbase + RW - RW % KB, RW % KB)])
            @pl.when(last)
            def _():
                pltpu.sync_copy(acc_sp.at[pl.ds(RW * NS, N - RW * NS)],
                                out.at[chunk, pl.ds(RW * NS, N - RW * NS)])

    return agg


_edge_agg = {c: _build_edge_agg(c) for c in (8, 2)}


def _build_edge_attn(H):
    HPS = max(H // NC, 1)
    EB = E2 // NS
    G = EB // L
    mesh = plsc.VectorSubcoreMesh(core_axis_name="c", subcore_axis_name="s",
                                  num_cores=NC, num_subcores=NS)
    scr = ([pltpu.VMEM((N,), jnp.float32)] * (3 * HPS)
           + [pltpu.VMEM((EB,), jnp.int32)] * 2
           + [pltpu.VMEM((EB,), jnp.float32)] * HPS)

    @functools.partial(
        pl.kernel,
        out_type=(jax.ShapeDtypeStruct((H * E2,), jnp.float32),
                  jax.ShapeDtypeStruct((H * NS * N,), jnp.float32)),
        mesh=mesh,
        compiler_params=pltpu.CompilerParams(needs_layout_passes=False),
        scratch_types=scr,
    )
    def attn(asrc_f, adst_f, srcf, dstf, exw, denom_f, *scratch):
        as_t = scratch[0:HPS]
        ad_t = scratch[HPS:2 * HPS]
        den = scratch[2 * HPS:3 * HPS]
        src_v, dst_v = scratch[3 * HPS], scratch[3 * HPS + 1]
        exb = scratch[3 * HPS + 2:]
        cid = lax.axis_index("c")
        sid = lax.axis_index("s")

        for j in range(HPS):
            h = cid * HPS + j if H > 1 else 0
            off = pl.multiple_of(h * N, 8)
            pltpu.sync_copy(asrc_f.at[pl.ds(off, N)], as_t[j])
            pltpu.sync_copy(adst_f.at[pl.ds(off, N)], ad_t[j])
        ebase = sid * EB
        pltpu.sync_copy(srcf.at[pl.ds(ebase, EB)], src_v)
        pltpu.sync_copy(dstf.at[pl.ds(ebase, EB)], dst_v)

        def zero(i, _):
            for j in range(HPS):
                den[j][pl.ds(i * L, L)] = jnp.zeros((L,), jnp.float32)
            return 0
        lax.fori_loop(0, N // L, zero, 0)

        def grp(g, _):
            sv = src_v[pl.ds(g * L, L)]
            dv = dst_v[pl.ds(g * L, L)]
            ge = ebase + g * L + lax.iota(jnp.int32, L)
            valid = ge < E
            for j in range(HPS):
                a = plsc.load_gather(as_t[j], [sv])
                b = plsc.load_gather(ad_t[j], [dv])
                al = a + b
                al = jnp.where(al > 0, al, al * jnp.float32(0.2))
                e = jnp.where(valid, jnp.exp(al), jnp.float32(0.0))
                exb[j][pl.ds(g * L, L)] = e
                plsc.addupdate_scatter(den[j], [dv], e)
            return 0
        lax.fori_loop(0, G, grp, 0)

        def write():
            for j in range(HPS):
                h = cid * HPS + j if H > 1 else 0
                off = pl.multiple_of(h * E2 + ebase, 8)
                pltpu.sync_copy(exb[j], exw.at[pl.ds(off, EB)])
                doff = pl.multiple_of((h * NS + sid) * N, 8)
                pltpu.sync_copy(den[j], denom_f.at[pl.ds(doff, N)])
        if H == 1:
            pl.when(cid == 0)(write)
        else:
            write()

    return attn


_edge_attn = {h: _build_edge_attn(h) for h in (4, 1)}


def _attn_and_agg(hc, als, ald, srcf, dstf, src2, dst2, exs, heads):
    n = als.shape[0]
    exw, denom_f = _edge_attn[heads](
        als.T.reshape(heads * n), ald.T.reshape(heads * n), srcf, dstf)
    acc = _edge_agg[hc.shape[0]](hc, src2, dst2, exw)
    dent = denom_f.reshape(heads, NS, n).sum(axis=1).T + exs
    return acc, dent


def kernel(x, edge_index, W1, a_src1, a_dst1, b1, W2, a_src2, a_dst2, b2):
    src, dst = edge_index[0], edge_index[1]
    pad = E2 - E
    nb = E2 // NS // KB
    srcf = jnp.pad(src, (0, pad))
    dstf = jnp.pad(dst, (0, pad))
    src2 = srcf.reshape(NS, nb, KB)
    dst2 = dstf.reshape(NS, nb, KB)

    hc1, als1, ald1, exs1 = _proj(x, W1, a_src1, a_dst1)
    acc1, dent1 = _attn_and_agg(hc1, als1, ald1, srcf, dstf, src2, dst2,
                                exs1, HEADS)
    hc2, als2, ald2, exs2 = _mid(acc1, hc1, dent1, exs1, b1, W2,
                                 a_src2, a_dst2)
    acc2, dent2 = _attn_and_agg(hc2, als2, ald2, srcf, dstf, src2, dst2,
                                exs2, 1)
    return _final(acc2, hc2, dent2, exs2, b2)
